# Initial kernel scaffold; baseline (speedup 1.0000x reference)
import functools

import jax
import jax.numpy as jnp
from jax import lax
from jax.experimental import pallas as pl
from jax.experimental.pallas import tpu as pltpu
from jax.experimental.pallas import tpu_sc as plsc

N = 10000
E = 320000
IN_CH = 128
HEADS = 8
OUT_CH = 16
HC = HEADS * OUT_CH

NC = 2
NS = 16
NW = NC * NS
EPT = E // NW
RPT = N // NS
B = 80

_mesh = plsc.VectorSubcoreMesh(core_axis_name="c", subcore_axis_name="s")


def _stage_a_body(x_ref, w_ref, acat_w_ref, h_ref, a1_ref, a2_ref, m_ref):
    h = jnp.dot(x_ref[...], w_ref[...], preferred_element_type=jnp.float32)
    h_ref[...] = h
    a = jnp.dot(h, acat_w_ref[...], preferred_element_type=jnp.float32)
    a1_ref[...] = a
    a2_ref[...] = jnp.concatenate([a[:, 8:], a[:, :8]], axis=-1)
    cm = jnp.max(a, axis=0)
    m8 = jnp.maximum(cm[:8] + cm[8:], 0.0)
    m_ref[...] = jnp.concatenate([m8, m8]).reshape(1, 16)


def _stage_a(x, w, acat_w):
    return pl.pallas_call(
        _stage_a_body,
        out_shape=(
            jax.ShapeDtypeStruct((N, HC), jnp.float32),
            jax.ShapeDtypeStruct((N, 16), jnp.float32),
            jax.ShapeDtypeStruct((N, 16), jnp.float32),
            jax.ShapeDtypeStruct((1, 16), jnp.float32),
        ),
    )(x, w, acat_w)


def _pass1_body(a1_hbm, a2_hbm, src_hbm, dst_hbm, m_hbm,
                ex_hbm, dpart_hbm,
                src_v, dst_v, gs_v, gd_v, ex_v, m_v, den_sh, sem1, sem2):
    cid = lax.axis_index("c")
    sid = lax.axis_index("s")
    wid = cid * NS + sid

    @pl.loop(0, B)
    def _zero_buf(i):
        ex_v[i, :] = jnp.zeros((16,), jnp.float32)

    r0 = sid * RPT
    for k in range(RPT // B):
        pltpu.sync_copy(ex_v, den_sh.at[pl.ds(r0 + k * B, B)])
    rem = RPT - (RPT // B) * B
    pltpu.sync_copy(ex_v.at[pl.ds(0, rem)],
                    den_sh.at[pl.ds(r0 + (RPT // B) * B, rem)])
    plsc.subcore_barrier()

    pltpu.sync_copy(m_hbm, m_v)

    base0 = wid * EPT

    @pl.loop(0, EPT, step=B)
    def _chunk(off):
        base = base0 + off
        pltpu.sync_copy(src_hbm.at[pl.ds(base, B)], src_v)
        pltpu.sync_copy(dst_hbm.at[pl.ds(base, B)], dst_v)
        cp1 = pltpu.async_copy(a1_hbm.at[src_v], gs_v, sem1)
        cp2 = pltpu.async_copy(a2_hbm.at[dst_v], gd_v, sem2)
        cp1.wait()
        cp2.wait()
        mvec = m_v[...]

        @pl.loop(0, B)
        def _edge(e):
            raw = gs_v[e, :] + gd_v[e, :]
            al = jnp.maximum(raw, raw * 0.2)
            ex_v[e, :] = jnp.exp(al - mvec)

        pltpu.sync_copy(ex_v, ex_hbm.at[pl.ds(base, B)])
        pltpu.sync_copy(ex_v, den_sh.at[dst_v], add=True)

    plsc.subcore_barrier()
    for k in range(RPT // B):
        pltpu.sync_copy(den_sh.at[pl.ds(r0 + k * B, B)], gs_v)
        pltpu.sync_copy(gs_v, dpart_hbm.at[cid, pl.ds(r0 + k * B, B)])
    pltpu.sync_copy(den_sh.at[pl.ds(r0 + (RPT // B) * B, rem)],
                    gs_v.at[pl.ds(0, rem)])
    pltpu.sync_copy(gs_v.at[pl.ds(0, rem)],
                    dpart_hbm.at[cid, pl.ds(r0 + (RPT // B) * B, rem)])


def _pass1(a1, a2, src, dst, m):
    f = pl.kernel(
        _pass1_body,
        out_type=(
            jax.ShapeDtypeStruct((E, 16), jnp.float32),
            jax.ShapeDtypeStruct((NC, N, 16), jnp.float32),
        ),
        mesh=_mesh,
        scratch_types=[
            pltpu.VMEM((B,), jnp.int32),
            pltpu.VMEM((B,), jnp.int32),
            pltpu.VMEM((B, 16), jnp.float32),
            pltpu.VMEM((B, 16), jnp.float32),
            pltpu.VMEM((B, 16), jnp.float32),
            pltpu.VMEM((16,), jnp.float32),
            pltpu.VMEM_SHARED((N, 16), jnp.float32),
            pltpu.SemaphoreType.DMA,
            pltpu.SemaphoreType.DMA,
        ],
    )
    return f(a1, a2, src, dst, m)


def _pass2_body(h_hbm, src_hbm, dst_hbm, ex_hbm, d0_hbm, d1_hbm,
                opart_hbm,
                src_v, dst_v, ex_v, d0_v, d1_v, at_v, rows_v,
                out_sh, sem1, sem2, sem3):
    cid = lax.axis_index("c")
    sid = lax.axis_index("s")
    wid = cid * NS + sid

    @pl.loop(0, B)
    def _zero_buf(i):
        for j in range(HEADS):
            rows_v[i, pl.ds(16 * j, 16)] = jnp.zeros((16,), jnp.float32)

    r0 = sid * RPT
    for k in range(RPT // B):
        pltpu.sync_copy(rows_v, out_sh.at[pl.ds(r0 + k * B, B)])
    rem = RPT - (RPT // B) * B
    pltpu.sync_copy(rows_v.at[pl.ds(0, rem)],
                    out_sh.at[pl.ds(r0 + (RPT // B) * B, rem)])
    plsc.subcore_barrier()

    base0 = wid * EPT

    @pl.loop(0, EPT, step=B)
    def _chunk(off):
        base = base0 + off
        pltpu.sync_copy(src_hbm.at[pl.ds(base, B)], src_v)
        pltpu.sync_copy(dst_hbm.at[pl.ds(base, B)], dst_v)
        cp1 = pltpu.async_copy(h_hbm.at[src_v], rows_v, sem1)
        cp2 = pltpu.async_copy(d0_hbm.at[dst_v], d0_v, sem2)
        cp3 = pltpu.async_copy(d1_hbm.at[dst_v], d1_v, sem3)
        pltpu.sync_copy(ex_hbm.at[pl.ds(base, B)], ex_v)
        cp1.wait()
        cp2.wait()
        cp3.wait()

        @pl.loop(0, B)
        def _edge(e):
            at_v[e, :] = ex_v[e, :] / (d0_v[e, :] + d1_v[e, :] + 1e-16)
            for j in range(HEADS):
                rows_v[e, pl.ds(16 * j, 16)] = (
                    rows_v[e, pl.ds(16 * j, 16)] * at_v[e, j])

        pltpu.sync_copy(rows_v, out_sh.at[dst_v], add=True)

    plsc.subcore_barrier()
    for k in range(RPT // B):
        pltpu.sync_copy(out_sh.at[pl.ds(r0 + k * B, B)], rows_v)
        pltpu.sync_copy(rows_v, opart_hbm.at[cid, pl.ds(r0 + k * B, B)])
    pltpu.sync_copy(out_sh.at[pl.ds(r0 + (RPT // B) * B, rem)],
                    rows_v.at[pl.ds(0, rem)])
    pltpu.sync_copy(rows_v.at[pl.ds(0, rem)],
                    opart_hbm.at[cid, pl.ds(r0 + (RPT // B) * B, rem)])


def _pass2(h, src, dst, ex, d0, d1):
    f = pl.kernel(
        _pass2_body,
        out_type=jax.ShapeDtypeStruct((NC, N, HC), jnp.float32),
        mesh=_mesh,
        scratch_types=[
            pltpu.VMEM((B,), jnp.int32),
            pltpu.VMEM((B,), jnp.int32),
            pltpu.VMEM((B, 16), jnp.float32),
            pltpu.VMEM((B, 16), jnp.float32),
            pltpu.VMEM((B, 16), jnp.float32),
            pltpu.VMEM((B, 16), jnp.float32),
            pltpu.VMEM((B, HC), jnp.float32),
            pltpu.VMEM_SHARED((N, HC), jnp.float32),
            pltpu.SemaphoreType.DMA,
            pltpu.SemaphoreType.DMA,
            pltpu.SemaphoreType.DMA,
        ],
    )
    return f(h, src, dst, ex, d0, d1)


def _stage_c_body(p0_ref, p1_ref, b_ref, o_ref):
    s = p0_ref[...] + p1_ref[...] + b_ref[...]
    o_ref[...] = jnp.where(s > 0, s, jnp.expm1(jnp.minimum(s, 0.0)))


def _stage_c(p0, p1, b):
    return pl.pallas_call(
        _stage_c_body,
        out_shape=jax.ShapeDtypeStruct((N, HC), jnp.float32),
    )(p0, p1, b)


def kernel(x, edge_index, W, att_src, att_dst, bias):
    src = edge_index[0].astype(jnp.int32)
    dst = edge_index[1].astype(jnp.int32)

    acat_w = jnp.zeros((HC, 16), jnp.float32)
    for j in range(HEADS):
        acat_w = acat_w.at[16 * j:16 * j + 16, j].set(att_src[j])
        acat_w = acat_w.at[16 * j:16 * j + 16, 8 + j].set(att_dst[j])

    h, a1, a2, m = _stage_a(x, W, acat_w)
    ex, dpart = _pass1(a1, a2, src, dst, m.reshape(16))
    opart = _pass2(h, src, dst, ex, dpart[0], dpart[1])
    return _stage_c(opart[0], opart[1], bias.reshape(1, HC))


# speedup vs baseline: 46.8660x; 46.8660x over previous
import functools

import jax
import jax.numpy as jnp
from jax import lax
from jax.experimental import pallas as pl
from jax.experimental.pallas import tpu as pltpu
from jax.experimental.pallas import tpu_sc as plsc

N = 10000
E = 320000
IN_CH = 128
HEADS = 8
OUT_CH = 16
HC = HEADS * OUT_CH

NC = 2
NS = 16
NW = NC * NS
EPT = E // NW
NP = 10240
RPT = NP // NS
B = 80

_mesh = plsc.VectorSubcoreMesh(core_axis_name="c", subcore_axis_name="s")


def _stage_a_body(x_ref, w_ref, acat_w_ref, h_ref, a1_ref, a2_ref, m_ref):
    h = jnp.dot(x_ref[...], w_ref[...], preferred_element_type=jnp.float32)
    h_ref[...] = h
    a = jnp.dot(h, acat_w_ref[...], preferred_element_type=jnp.float32)
    a1_ref[...] = a
    a2_ref[...] = jnp.concatenate([a[:, 8:], a[:, :8]], axis=-1)
    cm = jnp.max(a, axis=0)
    m8 = jnp.maximum(cm[:8] + cm[8:], 0.0)
    m_ref[...] = jnp.concatenate([m8, m8]).reshape(1, 16)


def _stage_a(x, w, acat_w):
    return pl.pallas_call(
        _stage_a_body,
        out_shape=(
            jax.ShapeDtypeStruct((N, HC), jnp.float32),
            jax.ShapeDtypeStruct((N, 16), jnp.float32),
            jax.ShapeDtypeStruct((N, 16), jnp.float32),
            jax.ShapeDtypeStruct((1, 16), jnp.float32),
        ),
    )(x, w, acat_w)


def _pass1_body(a1_hbm, a2_hbm, src_hbm, dst_hbm, m_hbm,
                ex_hbm, dpart_hbm,
                src_v, dst_v, gs_v, gd_v, ex_v, m_v, den_sh, sem1, sem2):
    cid = lax.axis_index("c")
    sid = lax.axis_index("s")
    wid = cid * NS + sid

    @pl.loop(0, B)
    def _zero_buf(i):
        ex_v[i, :] = jnp.zeros((16,), jnp.float32)

    r0 = sid * RPT
    for k in range(RPT // B):
        pltpu.sync_copy(ex_v, den_sh.at[pl.ds(r0 + k * B, B)])
    plsc.subcore_barrier()

    pltpu.sync_copy(m_hbm, m_v)

    base0 = wid * EPT

    @pl.loop(0, EPT, step=B)
    def _chunk(off):
        base = base0 + off
        pltpu.sync_copy(src_hbm.at[pl.ds(base, B)], src_v)
        pltpu.sync_copy(dst_hbm.at[pl.ds(base, B)], dst_v)
        cp1 = pltpu.async_copy(a1_hbm.at[src_v], gs_v, sem1)
        cp2 = pltpu.async_copy(a2_hbm.at[dst_v], gd_v, sem2)
        cp1.wait()
        cp2.wait()
        mvec = m_v[...]

        @pl.loop(0, B)
        def _edge(e):
            raw = gs_v[e, :] + gd_v[e, :]
            al = jnp.maximum(raw, raw * 0.2)
            ex_v[e, :] = jnp.exp(al - mvec)

        pltpu.sync_copy(ex_v, ex_hbm.at[pl.ds(base, B)])
        pltpu.sync_copy(ex_v, den_sh.at[dst_v], add=True)

    plsc.subcore_barrier()
    for k in range(RPT // B):
        pltpu.sync_copy(den_sh.at[pl.ds(r0 + k * B, B)], gs_v)
        pltpu.sync_copy(gs_v, dpart_hbm.at[cid, pl.ds(r0 + k * B, B)])


def _pass1(a1, a2, src, dst, m):
    f = pl.kernel(
        _pass1_body,
        out_type=(
            jax.ShapeDtypeStruct((E, 16), jnp.float32),
            jax.ShapeDtypeStruct((NC, NP, 16), jnp.float32),
        ),
        mesh=_mesh,
        compiler_params=pltpu.CompilerParams(use_tc_tiling_on_sc=False),
        scratch_types=[
            pltpu.VMEM((B,), jnp.int32),
            pltpu.VMEM((B,), jnp.int32),
            pltpu.VMEM((B, 16), jnp.float32),
            pltpu.VMEM((B, 16), jnp.float32),
            pltpu.VMEM((B, 16), jnp.float32),
            pltpu.VMEM((16,), jnp.float32),
            pltpu.VMEM_SHARED((NP, 16), jnp.float32),
            pltpu.SemaphoreType.DMA,
            pltpu.SemaphoreType.DMA,
        ],
    )
    return f(a1, a2, src, dst, m)


def _pass2_body(h_hbm, src_hbm, dst_hbm, ex_hbm, d0_hbm, d1_hbm,
                opart_hbm,
                src_v, dst_v, ex_v, d0_v, d1_v, rows_v,
                out_sh, sem1, sem2, sem3):
    cid = lax.axis_index("c")
    sid = lax.axis_index("s")
    wid = cid * NS + sid

    @pl.loop(0, B)
    def _zero_buf(i):
        for j in range(HEADS):
            rows_v[i, pl.ds(16 * j, 16)] = jnp.zeros((16,), jnp.float32)

    r0 = sid * RPT
    for k in range(RPT // B):
        pltpu.sync_copy(rows_v, out_sh.at[pl.ds(r0 + k * B, B)])
    plsc.subcore_barrier()

    base0 = wid * EPT

    @pl.loop(0, EPT, step=B)
    def _chunk(off):
        base = base0 + off
        pltpu.sync_copy(src_hbm.at[pl.ds(base, B)], src_v)
        pltpu.sync_copy(dst_hbm.at[pl.ds(base, B)], dst_v)
        cp1 = pltpu.async_copy(h_hbm.at[src_v], rows_v, sem1)
        cp2 = pltpu.async_copy(d0_hbm.at[dst_v], d0_v, sem2)
        cp3 = pltpu.async_copy(d1_hbm.at[dst_v], d1_v, sem3)
        pltpu.sync_copy(ex_hbm.at[pl.ds(base, B)], ex_v)
        cp1.wait()
        cp2.wait()
        cp3.wait()

        @pl.loop(0, B)
        def _edge(e):
            av = ex_v[e, :] / (d0_v[e, :] + d1_v[e, :] + 1e-16)
            for j in range(HEADS):
                rows_v[e, pl.ds(16 * j, 16)] = (
                    rows_v[e, pl.ds(16 * j, 16)] * av[j])

        pltpu.sync_copy(rows_v, out_sh.at[dst_v], add=True)

    plsc.subcore_barrier()
    for k in range(RPT // B):
        pltpu.sync_copy(out_sh.at[pl.ds(r0 + k * B, B)], rows_v)
        pltpu.sync_copy(rows_v, opart_hbm.at[cid, pl.ds(r0 + k * B, B)])


def _pass2(h, src, dst, ex, d0, d1):
    f = pl.kernel(
        _pass2_body,
        out_type=jax.ShapeDtypeStruct((NC, NP, HC), jnp.float32),
        mesh=_mesh,
        compiler_params=pltpu.CompilerParams(use_tc_tiling_on_sc=False),
        scratch_types=[
            pltpu.VMEM((B,), jnp.int32),
            pltpu.VMEM((B,), jnp.int32),
            pltpu.VMEM((B, 16), jnp.float32),
            pltpu.VMEM((B, 16), jnp.float32),
            pltpu.VMEM((B, 16), jnp.float32),
            pltpu.VMEM((B, HC), jnp.float32),
            pltpu.VMEM_SHARED((NP, HC), jnp.float32),
            pltpu.SemaphoreType.DMA,
            pltpu.SemaphoreType.DMA,
            pltpu.SemaphoreType.DMA,
        ],
    )
    return f(h, src, dst, ex, d0, d1)


def _stage_c_body(p0_ref, p1_ref, b_ref, o_ref):
    s = p0_ref[...][:N] + p1_ref[...][:N] + b_ref[...]
    o_ref[...] = jnp.where(s > 0, s, jnp.exp(jnp.minimum(s, 0.0)) - 1.0)


def _stage_c(p0, p1, b):
    return pl.pallas_call(
        _stage_c_body,
        out_shape=jax.ShapeDtypeStruct((N, HC), jnp.float32),
    )(p0, p1, b)


def kernel(x, edge_index, W, att_src, att_dst, bias):
    src = edge_index[0].astype(jnp.int32)
    dst = edge_index[1].astype(jnp.int32)

    acat_w = jnp.zeros((HC, 16), jnp.float32)
    for j in range(HEADS):
        acat_w = acat_w.at[16 * j:16 * j + 16, j].set(att_src[j])
        acat_w = acat_w.at[16 * j:16 * j + 16, 8 + j].set(att_dst[j])

    h, a1, a2, m = _stage_a(x, W, acat_w)
    ex, dpart = _pass1(a1, a2, src, dst, m.reshape(16))
    opart = _pass2(h, src, dst, ex, dpart[0], dpart[1])
    return _stage_c(opart[0], opart[1], bias.reshape(1, HC))
